# baseline (speedup 1.0000x reference)
import functools

import jax
import jax.numpy as jnp
import numpy as np
from jax import lax
from jax.experimental import pallas as pl
from jax.experimental.pallas import tpu as pltpu
from jax.experimental.pallas import tpu_sc as plsc

NC, NS, LANES = 2, 16, 16
NW = NC * NS

KA, KB = 104, 96


def _sc_pooled_sums(x, tblw, D):
    B, L = x.shape
    W = tblw.shape[1]
    rows_w = B // NW
    nchunk = W // LANES
    x = x.reshape(-1)

    mesh = plsc.VectorSubcoreMesh(core_axis_name="c", subcore_axis_name="s")

    def body(x_hbm, tbl_hbm, out_hbm, idx_v, buf0, buf1, out_v, sem0, sem1):
        wid = lax.axis_index("s") * NC + lax.axis_index("c")
        base = wid * rows_w
        pltpu.sync_copy(x_hbm.at[pl.ds(base * L, rows_w * L)], idx_v)

        def start_row(b, buf, sem):
            pltpu.make_async_copy(
                tbl_hbm.at[idx_v.at[pl.ds(b * L, KA)]],
                buf.at[pl.ds(0, KA)], sem).start()
            pltpu.make_async_copy(
                tbl_hbm.at[idx_v.at[pl.ds(b * L + KA, KB)]],
                buf.at[pl.ds(KA, KB)], sem).start()

        def wait_row(b, buf, sem):
            pltpu.make_async_copy(
                tbl_hbm.at[idx_v.at[pl.ds(b * L, KA)]],
                buf.at[pl.ds(0, KA)], sem).wait()
            pltpu.make_async_copy(
                tbl_hbm.at[idx_v.at[pl.ds(b * L + KA, KB)]],
                buf.at[pl.ds(KA, KB)], sem).wait()

        def add_row(buf, r, acc):
            out = list(acc)
            for j in range(nchunk):
                w = buf[r, pl.ds(LANES * j, LANES)]
                ev = lax.bitcast_convert_type(w << 16, jnp.float32)
                od = lax.bitcast_convert_type(w & jnp.int32(-65536),
                                              jnp.float32)
                out[2 * j] = acc[2 * j] + ev
                out[2 * j + 1] = acc[2 * j + 1] + od
            return tuple(out)

        def reduce_row(b, buf):
            def rbody(r, acc):
                return add_row(buf, 2 * r + 1, add_row(buf, 2 * r, acc))

            acc = lax.fori_loop(
                0, L // 2, rbody,
                tuple(jnp.zeros((LANES,), jnp.float32)
                      for _ in range(2 * nchunk)))
            for j in range(2 * nchunk):
                out_v[b, pl.ds(LANES * j, LANES)] = acc[j]

        start_row(0, buf0, sem0)

        def pair(i, carry):
            b0 = 2 * i
            start_row(b0 + 1, buf1, sem1)
            wait_row(b0, buf0, sem0)
            reduce_row(b0, buf0)

            @pl.when(b0 + 2 < rows_w)
            def _():
                start_row(b0 + 2, buf0, sem0)

            wait_row(b0 + 1, buf1, sem1)
            reduce_row(b0 + 1, buf1)
            return carry

        lax.fori_loop(0, rows_w // 2, pair, 0)
        pltpu.sync_copy(out_v, out_hbm.at[pl.ds(base, rows_w)])

    return pl.kernel(
        body,
        out_type=jax.ShapeDtypeStruct((B, D), jnp.float32),
        mesh=mesh,
        scratch_types=[
            pltpu.VMEM((rows_w * L,), jnp.int32),
            pltpu.VMEM((L, W), jnp.int32),
            pltpu.VMEM((L, W), jnp.int32),
            pltpu.VMEM((rows_w, D), jnp.float32),
            pltpu.SemaphoreType.DMA,
            pltpu.SemaphoreType.DMA,
        ],
        compiler_params=pltpu.CompilerParams(use_tc_tiling_on_sc=False),
    )(x, tblw)


def _tc_mlp(sums, inv_len, W1, b1, W2, b2):
    B, D = sums.shape
    H = W1.shape[1]
    C = W2.shape[1]
    BLK = 512

    def body(s_ref, il_ref, w1_ref, b1_ref, w2_ref, b2_ref, o_ref):
        rep = s_ref[...] * il_ref[...]
        h = jnp.dot(rep, w1_ref[...], preferred_element_type=jnp.float32)
        h = jnp.maximum(h + b1_ref[...], 0.0)
        o_ref[...] = (jnp.dot(h, w2_ref[...], preferred_element_type=jnp.float32)
                      + b2_ref[...])

    grid = (B // BLK,)
    return pl.pallas_call(
        body,
        grid=grid,
        in_specs=[
            pl.BlockSpec((BLK, D), lambda i: (i, 0)),
            pl.BlockSpec((BLK, 1), lambda i: (i, 0)),
            pl.BlockSpec((D, H), lambda i: (0, 0)),
            pl.BlockSpec((1, H), lambda i: (0, 0)),
            pl.BlockSpec((H, C), lambda i: (0, 0)),
            pl.BlockSpec((1, C), lambda i: (0, 0)),
        ],
        out_specs=pl.BlockSpec((BLK, C), lambda i: (i, 0)),
        out_shape=jax.ShapeDtypeStruct((B, C), jnp.float32),
    )(sums, inv_len, W1, b1, W2, b2)


def _unpack_perm(D):
    perm = []
    for c in range(D):
        j, k = c // 32, c % 32
        perm.append(16 * j + k if k < 16 else D // 2 + 16 * j + (k - 16))
    return np.array(perm)


@jax.jit
def kernel(x, lengths, table, W1, b1, W2, b2):
    V, D = table.shape
    def rne(bits):
        return bits + jnp.int32(0x7FFF) + (
            lax.shift_right_logical(bits, 16) & jnp.int32(1))

    blo = lax.bitcast_convert_type(table[:, :D // 2], jnp.int32)
    bhi = lax.bitcast_convert_type(table[:, D // 2:], jnp.int32)
    tblw = (lax.shift_right_logical(rne(blo), 16)
            | (rne(bhi) & jnp.int32(-65536)))
    sums = _sc_pooled_sums(x, tblw, D)
    inv_len = (1.0 / lengths.astype(jnp.float32)).reshape(-1, 1)
    W1p = W1[_unpack_perm(D), :]
    return _tc_mlp(sums, inv_len, W1p, b1.reshape(1, -1), W2, b2.reshape(1, -1))


# speedup vs baseline: 1.2346x; 1.2346x over previous
import functools

import jax
import jax.numpy as jnp
import numpy as np
from jax import lax
from jax.experimental import pallas as pl
from jax.experimental.pallas import tpu as pltpu
from jax.experimental.pallas import tpu_sc as plsc

NC, NS, LANES = 2, 16, 16
NW = NC * NS

KA, KB = 104, 96


def _sc_pooled_sums(x, tbl, D, packed):
    B, L = x.shape
    W = tbl.shape[1]
    rows_w = B // NW
    nchunk = W // LANES
    x = x.reshape(-1)

    mesh = plsc.VectorSubcoreMesh(core_axis_name="c", subcore_axis_name="s")

    def body(x_hbm, tbl_hbm, out_hbm, idx_v, buf0, buf1, out_v, sem0, sem1):
        wid = lax.axis_index("s") * NC + lax.axis_index("c")
        base = wid * rows_w
        pltpu.sync_copy(x_hbm.at[pl.ds(base * L, rows_w * L)], idx_v)

        def start_row(b, buf, sem):
            pltpu.make_async_copy(
                tbl_hbm.at[idx_v.at[pl.ds(b * L, KA)]],
                buf.at[pl.ds(0, KA)], sem).start()
            pltpu.make_async_copy(
                tbl_hbm.at[idx_v.at[pl.ds(b * L + KA, KB)]],
                buf.at[pl.ds(KA, KB)], sem).start()

        def wait_row(b, buf, sem):
            pltpu.make_async_copy(
                tbl_hbm.at[idx_v.at[pl.ds(b * L, KA)]],
                buf.at[pl.ds(0, KA)], sem).wait()
            pltpu.make_async_copy(
                tbl_hbm.at[idx_v.at[pl.ds(b * L + KA, KB)]],
                buf.at[pl.ds(KA, KB)], sem).wait()

        def add_row(buf, r, acc):
            out = list(acc)
            if packed:
                for j in range(nchunk):
                    w = buf[r, pl.ds(LANES * j, LANES)]
                    ev = lax.bitcast_convert_type(w << 16, jnp.float32)
                    od = lax.bitcast_convert_type(w & jnp.int32(-65536),
                                                  jnp.float32)
                    out[2 * j] = acc[2 * j] + ev
                    out[2 * j + 1] = acc[2 * j + 1] + od
            else:
                for j in range(nchunk):
                    out[j] = acc[j] + buf[r, pl.ds(LANES * j, LANES)]
            return tuple(out)

        def reduce_row(b, buf):
            def rbody(r, acc):
                return add_row(buf, 2 * r + 1, add_row(buf, 2 * r, acc))

            acc = lax.fori_loop(
                0, L // 2, rbody,
                tuple(jnp.zeros((LANES,), jnp.float32)
                      for _ in range(D // LANES)))
            for j in range(D // LANES):
                out_v[b, pl.ds(LANES * j, LANES)] = acc[j]

        start_row(0, buf0, sem0)

        def pair(i, carry):
            b0 = 2 * i
            start_row(b0 + 1, buf1, sem1)
            wait_row(b0, buf0, sem0)
            reduce_row(b0, buf0)

            @pl.when(b0 + 2 < rows_w)
            def _():
                start_row(b0 + 2, buf0, sem0)

            wait_row(b0 + 1, buf1, sem1)
            reduce_row(b0 + 1, buf1)
            return carry

        lax.fori_loop(0, rows_w // 2, pair, 0)
        pltpu.sync_copy(out_v, out_hbm.at[pl.ds(base, rows_w)])

    dt = jnp.float32 if not packed else jnp.int32
    return pl.kernel(
        body,
        out_type=jax.ShapeDtypeStruct((B, D), jnp.float32),
        mesh=mesh,
        scratch_types=[
            pltpu.VMEM((rows_w * L,), jnp.int32),
            pltpu.VMEM((L, W), dt),
            pltpu.VMEM((L, W), dt),
            pltpu.VMEM((rows_w, D), jnp.float32),
            pltpu.SemaphoreType.DMA,
            pltpu.SemaphoreType.DMA,
        ],
        compiler_params=pltpu.CompilerParams(use_tc_tiling_on_sc=False),
    )(x, tbl)


def _tc_mlp(sums, inv_len, W1, b1, W2, b2):
    B, D = sums.shape
    H = W1.shape[1]
    C = W2.shape[1]
    BLK = 512

    def body(s_ref, il_ref, w1_ref, b1_ref, w2_ref, b2_ref, o_ref):
        rep = s_ref[...] * il_ref[...]
        h = jnp.dot(rep, w1_ref[...], preferred_element_type=jnp.float32)
        h = jnp.maximum(h + b1_ref[...], 0.0)
        o_ref[...] = (jnp.dot(h, w2_ref[...], preferred_element_type=jnp.float32)
                      + b2_ref[...])

    grid = (B // BLK,)
    return pl.pallas_call(
        body,
        grid=grid,
        in_specs=[
            pl.BlockSpec((BLK, D), lambda i: (i, 0)),
            pl.BlockSpec((BLK, 1), lambda i: (i, 0)),
            pl.BlockSpec((D, H), lambda i: (0, 0)),
            pl.BlockSpec((1, H), lambda i: (0, 0)),
            pl.BlockSpec((H, C), lambda i: (0, 0)),
            pl.BlockSpec((1, C), lambda i: (0, 0)),
        ],
        out_specs=pl.BlockSpec((BLK, C), lambda i: (i, 0)),
        out_shape=jax.ShapeDtypeStruct((B, C), jnp.float32),
    )(sums, inv_len, W1, b1, W2, b2)


def _unpack_perm(D):
    perm = []
    for c in range(D):
        j, k = c // 32, c % 32
        perm.append(16 * j + k if k < 16 else D // 2 + 16 * j + (k - 16))
    return np.array(perm)


@jax.jit
def kernel(x, lengths, table, W1, b1, W2, b2):
    B = x.shape[0]
    V, D = table.shape
    B1 = B // 2

    sumsA = _sc_pooled_sums(x[:B1], table, D, packed=False)

    lo = lax.bitcast_convert_type(
        table[:, :D // 2].astype(jnp.bfloat16), jnp.uint16).astype(jnp.uint32)
    hi = lax.bitcast_convert_type(
        table[:, D // 2:].astype(jnp.bfloat16), jnp.uint16).astype(jnp.uint32)
    tblw = lax.bitcast_convert_type(lo | (hi << 16), jnp.int32)

    sumsB = _sc_pooled_sums(x[B1:], tblw, D, packed=True)

    inv_len = (1.0 / lengths.astype(jnp.float32)).reshape(-1, 1)
    b1r, b2r = b1.reshape(1, -1), b2.reshape(1, -1)
    logitsA = _tc_mlp(sumsA, inv_len[:B1], W1, b1r, W2, b2r)
    logitsB = _tc_mlp(sumsB, inv_len[B1:], W1[_unpack_perm(D), :], b1r, W2, b2r)
    return jnp.concatenate([logitsA, logitsB], axis=0)


# speedup vs baseline: 1.3725x; 1.1117x over previous
import functools

import jax
import jax.numpy as jnp
import numpy as np
from jax import lax
from jax.experimental import pallas as pl
from jax.experimental.pallas import tpu as pltpu
from jax.experimental.pallas import tpu_sc as plsc

NC, NS, LANES = 2, 16, 16
NW = NC * NS

PARTS = ((0, 104), (104, 96))


def _sc_pooled_sums(x, tbl, D, packed, nbuf):
    B, L = x.shape
    W = tbl.shape[1]
    rows_w = B // NW
    nchunk = W // LANES
    rows_it = nbuf // 2
    assert rows_w % rows_it == 0
    x = x.reshape(-1)

    mesh = plsc.VectorSubcoreMesh(core_axis_name="c", subcore_axis_name="s")

    def body(x_hbm, tbl_hbm, out_hbm, idx_v, out_v, *bufsem):
        bufs, sems = bufsem[:nbuf], bufsem[nbuf:]
        wid = lax.axis_index("s") * NC + lax.axis_index("c")
        base = wid * rows_w
        pltpu.sync_copy(x_hbm.at[pl.ds(base * L, rows_w * L)], idx_v)

        def start_chunk(b, part, buf, sem):
            off, sz = PARTS[part]
            pltpu.make_async_copy(
                tbl_hbm.at[idx_v.at[pl.ds(b * L + off, sz)]], buf, sem
            ).start()

        def wait_chunk(b, part, buf, sem):
            off, sz = PARTS[part]
            pltpu.make_async_copy(
                tbl_hbm.at[idx_v.at[pl.ds(b * L + off, sz)]], buf, sem
            ).wait()

        def add_row(buf, r, acc):
            out = list(acc)
            if packed:
                for j in range(nchunk):
                    w = buf[r, pl.ds(LANES * j, LANES)]
                    ev = lax.bitcast_convert_type(w << 16, jnp.float32)
                    od = lax.bitcast_convert_type(w & jnp.int32(-65536),
                                                  jnp.float32)
                    out[2 * j] = acc[2 * j] + ev
                    out[2 * j + 1] = acc[2 * j + 1] + od
            else:
                for j in range(nchunk):
                    out[j] = acc[j] + buf[r, pl.ds(LANES * j, LANES)]
            return tuple(out)

        def reduce_chunk(buf, nrows, acc):
            def rbody(r, a):
                return add_row(buf, 2 * r + 1, add_row(buf, 2 * r, a))

            return lax.fori_loop(0, nrows // 2, rbody, acc)

        zeros = tuple(jnp.zeros((LANES,), jnp.float32)
                      for _ in range(D // LANES))

        for u in range(nbuf):
            start_chunk(u // 2, u % 2, bufs[u], sems[u])

        def step(i, carry):
            for u in range(nbuf):
                b = rows_it * i + u // 2
                part = u % 2
                wait_chunk(b, part, bufs[u], sems[u])

                @pl.when(b + rows_it < rows_w)
                def _():
                    start_chunk(b + rows_it, part, bufs[u], sems[u])

                if part == 0:
                    acc = reduce_chunk(bufs[u], PARTS[0][1], zeros)
                else:
                    acc = reduce_chunk(bufs[u], PARTS[1][1], acc)
                    for j in range(D // LANES):
                        out_v[b, pl.ds(LANES * j, LANES)] = acc[j]
            return carry

        lax.fori_loop(0, rows_w // rows_it, step, 0)
        pltpu.sync_copy(out_v, out_hbm.at[pl.ds(base, rows_w)])

    dt = jnp.float32 if not packed else jnp.int32
    scratch = [
        pltpu.VMEM((rows_w * L,), jnp.int32),
        pltpu.VMEM((rows_w, D), jnp.float32),
    ]
    scratch += [pltpu.VMEM((PARTS[u % 2][1], W), dt) for u in range(nbuf)]
    scratch += [pltpu.SemaphoreType.DMA for _ in range(nbuf)]
    return pl.kernel(
        body,
        out_type=jax.ShapeDtypeStruct((B, D), jnp.float32),
        mesh=mesh,
        scratch_types=scratch,
        compiler_params=pltpu.CompilerParams(use_tc_tiling_on_sc=False),
    )(x, tbl)


def _tc_mlp(sums, inv_len, W1, b1, W2, b2):
    B, D = sums.shape
    H = W1.shape[1]
    C = W2.shape[1]
    BLK = 512

    def body(s_ref, il_ref, w1_ref, b1_ref, w2_ref, b2_ref, o_ref):
        rep = s_ref[...] * il_ref[...]
        h = jnp.dot(rep, w1_ref[...], preferred_element_type=jnp.float32)
        h = jnp.maximum(h + b1_ref[...], 0.0)
        o_ref[...] = (jnp.dot(h, w2_ref[...], preferred_element_type=jnp.float32)
                      + b2_ref[...])

    grid = (B // BLK,)
    return pl.pallas_call(
        body,
        grid=grid,
        in_specs=[
            pl.BlockSpec((BLK, D), lambda i: (i, 0)),
            pl.BlockSpec((BLK, 1), lambda i: (i, 0)),
            pl.BlockSpec((D, H), lambda i: (0, 0)),
            pl.BlockSpec((1, H), lambda i: (0, 0)),
            pl.BlockSpec((H, C), lambda i: (0, 0)),
            pl.BlockSpec((1, C), lambda i: (0, 0)),
        ],
        out_specs=pl.BlockSpec((BLK, C), lambda i: (i, 0)),
        out_shape=jax.ShapeDtypeStruct((B, C), jnp.float32),
    )(sums, inv_len, W1, b1, W2, b2)


def _unpack_perm(D):
    perm = []
    for c in range(D):
        j, k = c // 32, c % 32
        perm.append(16 * j + k if k < 16 else D // 2 + 16 * j + (k - 16))
    return np.array(perm)


@jax.jit
def kernel(x, lengths, table, W1, b1, W2, b2):
    V, D = table.shape

    lo = lax.bitcast_convert_type(
        table[:, :D // 2].astype(jnp.bfloat16), jnp.uint16).astype(jnp.uint32)
    hi = lax.bitcast_convert_type(
        table[:, D // 2:].astype(jnp.bfloat16), jnp.uint16).astype(jnp.uint32)
    tblw = lax.bitcast_convert_type(lo | (hi << 16), jnp.int32)

    sums = _sc_pooled_sums(x, tblw, D, packed=True, nbuf=8)
    inv_len = (1.0 / lengths.astype(jnp.float32)).reshape(-1, 1)
    W1p = W1[_unpack_perm(D), :]
    return _tc_mlp(sums, inv_len, W1p, b1.reshape(1, -1), W2, b2.reshape(1, -1))


# speedup vs baseline: 1.7112x; 1.2467x over previous
import functools

import jax
import jax.numpy as jnp
import numpy as np
from jax import lax
from jax.experimental import pallas as pl
from jax.experimental.pallas import tpu as pltpu
from jax.experimental.pallas import tpu_sc as plsc

NC, NS, LANES = 2, 16, 16
NW = NC * NS

PARTS2 = ((0, 104), (104, 96))
PARTS4 = ((0, 56), (56, 48), (104, 48), (152, 48))


def _sc_pooled_sums(x, tbl, D, packed, nbuf, parts):
    B, L = x.shape
    W = tbl.shape[1]
    rows_w = B // NW
    nchunk = W // LANES
    ppr = len(parts)
    rows_it = nbuf // ppr
    assert nbuf % ppr == 0 and rows_w % rows_it == 0
    x = x.reshape(-1)

    mesh = plsc.VectorSubcoreMesh(core_axis_name="c", subcore_axis_name="s")

    def body(x_hbm, tbl_hbm, out_hbm, idx_v, out_v, *bufsem):
        bufs, sems = bufsem[:nbuf], bufsem[nbuf:]
        wid = lax.axis_index("s") * NC + lax.axis_index("c")
        base = wid * rows_w
        pltpu.sync_copy(x_hbm.at[pl.ds(base * L, rows_w * L)], idx_v)

        def start_chunk(b, part, buf, sem):
            off, sz = parts[part]
            pltpu.make_async_copy(
                tbl_hbm.at[idx_v.at[pl.ds(b * L + off, sz)]], buf, sem
            ).start()

        def wait_chunk(b, part, buf, sem):
            off, sz = parts[part]
            pltpu.make_async_copy(
                tbl_hbm.at[idx_v.at[pl.ds(b * L + off, sz)]], buf, sem
            ).wait()

        def add_row(buf, r, acc):
            out = list(acc)
            if packed:
                for j in range(nchunk):
                    w = buf[r, pl.ds(LANES * j, LANES)]
                    ev = lax.bitcast_convert_type(w << 16, jnp.float32)
                    od = lax.bitcast_convert_type(w & jnp.int32(-65536),
                                                  jnp.float32)
                    out[2 * j] = acc[2 * j] + ev
                    out[2 * j + 1] = acc[2 * j + 1] + od
            else:
                for j in range(nchunk):
                    out[j] = acc[j] + buf[r, pl.ds(LANES * j, LANES)]
            return tuple(out)

        def reduce_chunk(buf, nrows, acc):
            def rbody(r, a):
                return add_row(buf, 2 * r + 1, add_row(buf, 2 * r, a))

            return lax.fori_loop(0, nrows // 2, rbody, acc)

        zeros = tuple(jnp.zeros((LANES,), jnp.float32)
                      for _ in range(D // LANES))

        for u in range(nbuf):
            start_chunk(u // ppr, u % ppr, bufs[u], sems[u])

        def step(i, carry):
            for u in range(nbuf):
                b = rows_it * i + u // ppr
                part = u % ppr
                wait_chunk(b, part, bufs[u], sems[u])

                @pl.when(b + rows_it < rows_w)
                def _():
                    start_chunk(b + rows_it, part, bufs[u], sems[u])

                acc = reduce_chunk(bufs[u], parts[part][1],
                                   zeros if part == 0 else acc)
                if part == ppr - 1:
                    for j in range(D // LANES):
                        out_v[b, pl.ds(LANES * j, LANES)] = acc[j]
            return carry

        lax.fori_loop(0, rows_w // rows_it, step, 0)
        pltpu.sync_copy(out_v, out_hbm.at[pl.ds(base, rows_w)])

    dt = jnp.float32 if not packed else jnp.int32
    scratch = [
        pltpu.VMEM((rows_w * L,), jnp.int32),
        pltpu.VMEM((rows_w, D), jnp.float32),
    ]
    scratch += [pltpu.VMEM((parts[u % ppr][1], W), dt) for u in range(nbuf)]
    scratch += [pltpu.SemaphoreType.DMA for _ in range(nbuf)]
    return pl.kernel(
        body,
        out_type=jax.ShapeDtypeStruct((B, D), jnp.float32),
        mesh=mesh,
        scratch_types=scratch,
        compiler_params=pltpu.CompilerParams(use_tc_tiling_on_sc=False),
    )(x, tbl)


def _tc_mlp(sums, inv_len, W1, b1, W2, b2):
    B, D = sums.shape
    H = W1.shape[1]
    C = W2.shape[1]
    BLK = 512

    def body(s_ref, il_ref, w1_ref, b1_ref, w2_ref, b2_ref, o_ref):
        rep = s_ref[...] * il_ref[...]
        h = jnp.dot(rep, w1_ref[...], preferred_element_type=jnp.float32)
        h = jnp.maximum(h + b1_ref[...], 0.0)
        o_ref[...] = (jnp.dot(h, w2_ref[...], preferred_element_type=jnp.float32)
                      + b2_ref[...])

    grid = (B // BLK,)
    return pl.pallas_call(
        body,
        grid=grid,
        in_specs=[
            pl.BlockSpec((BLK, D), lambda i: (i, 0)),
            pl.BlockSpec((BLK, 1), lambda i: (i, 0)),
            pl.BlockSpec((D, H), lambda i: (0, 0)),
            pl.BlockSpec((1, H), lambda i: (0, 0)),
            pl.BlockSpec((H, C), lambda i: (0, 0)),
            pl.BlockSpec((1, C), lambda i: (0, 0)),
        ],
        out_specs=pl.BlockSpec((BLK, C), lambda i: (i, 0)),
        out_shape=jax.ShapeDtypeStruct((B, C), jnp.float32),
    )(sums, inv_len, W1, b1, W2, b2)


def _unpack_perm(D):
    perm = []
    for c in range(D):
        j, k = c // 32, c % 32
        perm.append(16 * j + k if k < 16 else D // 2 + 16 * j + (k - 16))
    return np.array(perm)


@jax.jit
def kernel(x, lengths, table, W1, b1, W2, b2):
    V, D = table.shape
    sums = _sc_pooled_sums(x, table, D, packed=False, nbuf=8, parts=PARTS4)
    inv_len = (1.0 / lengths.astype(jnp.float32)).reshape(-1, 1)
    return _tc_mlp(sums, inv_len, W1, b1.reshape(1, -1), W2, b2.reshape(1, -1))
